# baseline (speedup 1.0000x reference)
import functools

import jax
import jax.numpy as jnp
from jax import lax
from jax.experimental import pallas as pl
from jax.experimental.pallas import tpu as pltpu
from jax.experimental.pallas import tpu_sc as plsc

B = 4096
L = 200
D = 32
V = 1000000
HALF = L // 2

NC, NS = 2, 16
NW = NC * NS
ROWS_PER_W = B // NW
HALVES_PER_W = ROWS_PER_W * 2
NBUF = 4

NFULL = V // 128
VFULL = NFULL * 128
K1_MAX = -(-NFULL // NW) + 1

_mesh = plsc.VectorSubcoreMesh(core_axis_name="c", subcore_axis_name="s")


@functools.partial(
    pl.kernel,
    mesh=_mesh,
    out_type=jax.ShapeDtypeStruct((V * D,), jnp.float32),
    compiler_params=pltpu.CompilerParams(use_tc_tiling_on_sc=True,
                                         needs_layout_passes=False),
    scratch_types=[pltpu.VMEM((D, 128), jnp.float32)] * 4
    + [pltpu.VMEM((128 * D,), jnp.float32)] * 4
    + [pltpu.SemaphoreType.DMA] * 8,
)
def _transpose(wt_hbm, wtail_hbm, out_hbm, *refs):
    wid = lax.axis_index("s") * NC + lax.axis_index("c")
    in_refs = refs[0:4]
    st_refs = refs[4:8]
    sin = refs[8:12]
    sout = refs[12:16]
    def blk(k):
        return wid + NW * k

    def in_copy(k, s):
        b = blk(k)
        return pltpu.make_async_copy(
            wt_hbm.at[pl.ds(0, D), pl.ds(b * 128, 128)], in_refs[s], sin[s])

    def out_copy(k, s):
        b = blk(k)
        return pltpu.make_async_copy(
            st_refs[s], out_hbm.at[pl.ds(b * 128 * D, 128 * D)], sout[s])

    lane = lax.iota(jnp.int32, 16)

    def transpose_block(s, ntok):
        def tb(t8, _):
            t0 = t8 * 16
            tvec = lane + t0
            t32 = tvec * D
            for d0 in range(D):
                c = lane ^ d0
                v = plsc.load_gather(in_refs[s], [c, tvec])
                plsc.store_scatter(st_refs[s], [t32 + c], v)
            return 0
        lax.fori_loop(0, ntok // 16, tb, 0, unroll=2)

    def start_in(k, s):
        @pl.when(blk(k) < NFULL)
        def _():
            in_copy(k, s).start()

    for s in range(2):
        start_in(s, s)

    @pl.loop(0, K1_MAX, step=2)
    def _(k):
        for s in range(2):
            kk = k + s

            @pl.when(blk(kk) < NFULL)
            def _():
                in_copy(kk, s).wait()

                @pl.when(kk >= 2)
                def _():
                    out_copy(kk - 2, s).wait()

                transpose_block(s, 128)
                out_copy(kk, s).start()

            start_in(kk + 2, s)

    @pl.loop(0, K1_MAX, step=2)
    def _(k):
        for s in range(2):
            kk = k + s

            @pl.when((blk(kk) < NFULL) & (blk(kk) + 2 * NW >= NFULL))
            def _():
                out_copy(kk, s).wait()

    @pl.when(wid == 0)
    def _():
        pltpu.sync_copy(wtail_hbm, in_refs[0])
        transpose_block(0, V - VFULL)
        pltpu.sync_copy(
            st_refs[0].at[pl.ds(0, (V - VFULL) * D)],
            out_hbm.at[pl.ds(VFULL * D, (V - VFULL) * D)])


@functools.partial(
    pl.kernel,
    mesh=_mesh,
    out_type=jax.ShapeDtypeStruct((B, D), jnp.float32),
    compiler_params=pltpu.CompilerParams(use_tc_tiling_on_sc=False),
    scratch_types=[
        pltpu.VMEM((HALVES_PER_W, HALF), jnp.int32),
        pltpu.VMEM((NBUF, HALF, D), jnp.float32),
        pltpu.VMEM((ROWS_PER_W, D), jnp.float32),
        pltpu.VMEM((D,), jnp.float32),
    ] + [pltpu.SemaphoreType.DMA] * NBUF,
)
def _emb_sum(x_hbm, w_hbm, b_hbm, out_hbm, idx_v, buf_v, out_v, bias_v,
             *sems):
    wid = lax.axis_index("s") * NC + lax.axis_index("c")
    base = wid * HALVES_PER_W

    pltpu.sync_copy(x_hbm.at[pl.ds(base, HALVES_PER_W)], idx_v)
    pltpu.sync_copy(b_hbm, bias_v)
    bias0 = bias_v[pl.ds(0, 16)]
    bias1 = bias_v[pl.ds(16, 16)]

    def start(i, slot):
        pltpu.async_copy(w_hbm.at[idx_v.at[i]], buf_v.at[slot], sems[slot])

    def wait(i, slot):
        pltpu.make_async_copy(
            w_hbm.at[idx_v.at[i]], buf_v.at[slot], sems[slot]).wait()

    def accum(slot, accs):
        def inner(j, accs):
            a0, a1 = accs
            a0 = a0 + buf_v[slot, j, pl.ds(0, 16)]
            a1 = a1 + buf_v[slot, j, pl.ds(16, 16)]
            return (a0, a1)
        return lax.fori_loop(0, HALF, inner, accs, unroll=10)

    for slot in range(NBUF):
        start(slot, slot)

    @pl.loop(0, HALVES_PER_W, step=NBUF)
    def _(g):
        for half in range(NBUF // 2):
            accs = (bias0, bias1)
            for s2 in range(2):
                slot = half * 2 + s2
                wait(g + slot, slot)
                accs = accum(slot, accs)

                @pl.when(g < HALVES_PER_W - NBUF)
                def _():
                    start(g + slot + NBUF, slot)

            row = g // 2 + half
            out_v[row, pl.ds(0, 16)] = accs[0]
            out_v[row, pl.ds(16, 16)] = accs[1]

    pltpu.sync_copy(out_v, out_hbm.at[pl.ds(wid * ROWS_PER_W, ROWS_PER_W)])


def kernel(x, emb_weight, emb_bias):
    wt = emb_weight.T
    wtail = jnp.pad(emb_weight[VFULL:], ((0, 128 - (V - VFULL)), (0, 0))).T
    wlin = _transpose(wt, wtail)
    w2 = wlin.reshape(V, D)
    x2 = x.reshape(B * 2, HALF)
    return _emb_sum(x2, w2, emb_bias)


# speedup vs baseline: 1.0423x; 1.0423x over previous
import functools

import jax
import jax.numpy as jnp
from jax import lax
from jax.experimental import pallas as pl
from jax.experimental.pallas import tpu as pltpu
from jax.experimental.pallas import tpu_sc as plsc

B = 4096
L = 200
D = 32
V = 1000000
HALF = L // 2

NC, NS = 2, 16
NW = NC * NS
ROWS_PER_W = B // NW
HALVES_PER_W = ROWS_PER_W * 2
NBUF = 4

NFULL = V // 128
VFULL = NFULL * 128
K1_MAX = 246

_mesh = plsc.VectorSubcoreMesh(core_axis_name="c", subcore_axis_name="s")


@functools.partial(
    pl.kernel,
    mesh=_mesh,
    out_type=jax.ShapeDtypeStruct((V * D,), jnp.float32),
    compiler_params=pltpu.CompilerParams(use_tc_tiling_on_sc=True,
                                         needs_layout_passes=False,
                                         disable_bounds_checks=True),
    scratch_types=[pltpu.VMEM((D, 128), jnp.float32)] * 4
    + [pltpu.VMEM((128 * D,), jnp.float32)] * 4
    + [pltpu.SemaphoreType.DMA] * 8,
)
def _transpose(wt_hbm, wtail_hbm, out_hbm, *refs):
    wid = lax.axis_index("s") * NC + lax.axis_index("c")
    in_refs = refs[0:4]
    st_refs = refs[4:8]
    sin = refs[8:12]
    sout = refs[12:16]
    def blk(k):
        return wid + NW * k

    def in_copy(k, s):
        b = blk(k)
        return pltpu.make_async_copy(
            wt_hbm.at[pl.ds(0, D), pl.ds(b * 128, 128)], in_refs[s], sin[s])

    def out_copy(k, s):
        b = blk(k)
        return pltpu.make_async_copy(
            st_refs[s], out_hbm.at[pl.ds(b * 128 * D, 128 * D)], sout[s])

    lane = lax.iota(jnp.int32, 16)

    def transpose_block(s, ntok):
        def tb(t8, _):
            t0 = t8 * 16
            tvec = lane + t0
            t32 = tvec * D
            for d0 in range(D):
                c = lane ^ d0
                v = plsc.load_gather(in_refs[s], [c, tvec])
                plsc.store_scatter(st_refs[s], [t32 + c], v)
            return 0
        lax.fori_loop(0, ntok // 16, tb, 0)

    def start_in(k, s):
        @pl.when(blk(k) < NFULL)
        def _():
            in_copy(k, s).start()

    for s in range(3):
        start_in(s, s)

    @pl.loop(0, K1_MAX, step=3)
    def _(k):
        for s in range(3):
            kk = k + s

            @pl.when(blk(kk) < NFULL)
            def _():
                in_copy(kk, s).wait()

                @pl.when(kk >= 3)
                def _():
                    out_copy(kk - 3, s).wait()

                transpose_block(s, 128)
                out_copy(kk, s).start()

            start_in(kk + 3, s)

    @pl.loop(0, K1_MAX, step=3)
    def _(k):
        for s in range(3):
            kk = k + s

            @pl.when((blk(kk) < NFULL) & (blk(kk) + 3 * NW >= NFULL))
            def _():
                out_copy(kk, s).wait()

    @pl.when(wid == 0)
    def _():
        pltpu.sync_copy(wtail_hbm, in_refs[0])
        transpose_block(0, V - VFULL)
        pltpu.sync_copy(
            st_refs[0].at[pl.ds(0, (V - VFULL) * D)],
            out_hbm.at[pl.ds(VFULL * D, (V - VFULL) * D)])


@functools.partial(
    pl.kernel,
    mesh=_mesh,
    out_type=jax.ShapeDtypeStruct((B, D), jnp.float32),
    compiler_params=pltpu.CompilerParams(use_tc_tiling_on_sc=False,
                                         disable_bounds_checks=True),
    scratch_types=[
        pltpu.VMEM((HALVES_PER_W, HALF), jnp.int32),
        pltpu.VMEM((NBUF, HALF, D), jnp.float32),
        pltpu.VMEM((ROWS_PER_W, D), jnp.float32),
        pltpu.VMEM((D,), jnp.float32),
    ] + [pltpu.SemaphoreType.DMA] * NBUF,
)
def _emb_sum(x_hbm, w_hbm, b_hbm, out_hbm, idx_v, buf_v, out_v, bias_v,
             *sems):
    wid = lax.axis_index("s") * NC + lax.axis_index("c")
    base = wid * HALVES_PER_W

    pltpu.sync_copy(x_hbm.at[pl.ds(base, HALVES_PER_W)], idx_v)
    pltpu.sync_copy(b_hbm, bias_v)
    bias0 = bias_v[pl.ds(0, 16)]
    bias1 = bias_v[pl.ds(16, 16)]

    def start(i, slot):
        pltpu.async_copy(w_hbm.at[idx_v.at[i]], buf_v.at[slot], sems[slot])

    def wait(i, slot):
        pltpu.make_async_copy(
            w_hbm.at[idx_v.at[i]], buf_v.at[slot], sems[slot]).wait()

    def accum(slot, accs):
        def inner(j, accs):
            a0, a1 = accs
            a0 = a0 + buf_v[slot, j, pl.ds(0, 16)]
            a1 = a1 + buf_v[slot, j, pl.ds(16, 16)]
            return (a0, a1)
        return lax.fori_loop(0, HALF, inner, accs, unroll=10)

    for slot in range(NBUF):
        start(slot, slot)

    @pl.loop(0, HALVES_PER_W, step=NBUF)
    def _(g):
        for half in range(NBUF // 2):
            accs = (bias0, bias1)
            for s2 in range(2):
                slot = half * 2 + s2
                wait(g + slot, slot)
                accs = accum(slot, accs)

                @pl.when(g < HALVES_PER_W - NBUF)
                def _():
                    start(g + slot + NBUF, slot)

            row = g // 2 + half
            out_v[row, pl.ds(0, 16)] = accs[0]
            out_v[row, pl.ds(16, 16)] = accs[1]

    pltpu.sync_copy(out_v, out_hbm.at[pl.ds(wid * ROWS_PER_W, ROWS_PER_W)])


def kernel(x, emb_weight, emb_bias):
    wt = emb_weight.T
    wtail = jnp.pad(emb_weight[VFULL:], ((0, 128 - (V - VFULL)), (0, 0))).T
    wlin = _transpose(wt, wtail)
    w2 = wlin.reshape(V, D)
    x2 = x.reshape(B * 2, HALF)
    return _emb_sum(x2, w2, emb_bias)
